# Initial kernel scaffold; baseline (speedup 1.0000x reference)
import jax
import jax.numpy as jnp
from jax.experimental import pallas as pl


def kernel(z_fast_real, z_fast_imag, z_slow_real, z_slow_imag, cb_syn, cb_sem, W_ctx_syn, b_ctx_syn, W_ctx_sem, b_ctx_sem):
    raise NotImplementedError("write your pallas kernel here")


# speedup vs baseline: 1.0086x; 1.0086x over previous
import functools

import jax
import jax.numpy as jnp
from jax.experimental import pallas as pl

B = 16384
D = 128
DIM = 2 * D
N_SYN = 512
N_SEM = 1024
CTX_GATE_STRENGTH = 2.0
COMMITMENT_COST = 0.25

BLOCK_B = 512


def _vq_block(z, zsq, cbT, cb, WT, b, csq):
    k = cb.shape[0]
    zc = jax.lax.dot_general(z, cbT, (((1,), (0,)), ((), ())),
                             preferred_element_type=jnp.float32)
    d = (zsq + csq) - 2.0 * zc
    logits = jax.lax.dot_general(z, WT, (((1,), (0,)), ((), ())),
                                 preferred_element_type=jnp.float32) + b
    m = jnp.max(logits, axis=1, keepdims=True)
    e = jnp.exp(logits - m)
    bias = CTX_GATE_STRENGTH * (e / jnp.sum(e, axis=1, keepdims=True))
    dtot = d - bias
    dmin = jnp.min(dtot, axis=1, keepdims=True)
    lane = jax.lax.broadcasted_iota(jnp.int32, dtot.shape, 1)
    idx = jnp.min(jnp.where(dtot == dmin, lane, k), axis=1)
    onehot = (lane == idx[:, None]).astype(jnp.float32)
    zq = jax.lax.dot_general(onehot, cb, (((1,), (0,)), ((), ())),
                             preferred_element_type=jnp.float32,
                             precision=jax.lax.Precision.HIGHEST)
    r = zq - z
    return zq, idx, jnp.sum(r * r)


def _fused_kernel(zf_ref, zs_ref, zfsq_ref, zssq_ref,
                  cbT_syn_ref, cb_syn_ref, WT_syn_ref, b_syn_ref, csq_syn_ref,
                  cbT_sem_ref, cb_sem_ref, WT_sem_ref, b_sem_ref, csq_sem_ref,
                  qf_ref, qs_ref, idx_syn_ref, idx_sem_ref, loss_ref):
    qf, i_syn, l_syn = _vq_block(zf_ref[...], zfsq_ref[...],
                                 cbT_syn_ref[...], cb_syn_ref[...],
                                 WT_syn_ref[...], b_syn_ref[...], csq_syn_ref[...])
    qs, i_sem, l_sem = _vq_block(zs_ref[...], zssq_ref[...],
                                 cbT_sem_ref[...], cb_sem_ref[...],
                                 WT_sem_ref[...], b_sem_ref[...], csq_sem_ref[...])
    qf_ref[...] = qf
    qs_ref[...] = qs
    idx_syn_ref[...] = i_syn[:, None]
    idx_sem_ref[...] = i_sem[:, None]

    @pl.when(pl.program_id(0) == 0)
    def _init():
        loss_ref[...] = jnp.zeros_like(loss_ref)

    loss_ref[...] += l_syn + l_sem


@functools.partial(jax.jit, static_argnames=())
def kernel(z_fast_real, z_fast_imag, z_slow_real, z_slow_imag,
           cb_syn, cb_sem, W_ctx_syn, b_ctx_syn, W_ctx_sem, b_ctx_sem):
    zf = jnp.concatenate([z_fast_real, z_fast_imag], axis=1)
    zs = jnp.concatenate([z_slow_real, z_slow_imag], axis=1)
    cbT_syn = cb_syn.T
    cbT_sem = cb_sem.T
    WT_syn = W_ctx_syn.T
    WT_sem = W_ctx_sem.T
    csq_syn = jnp.sum(cb_syn ** 2, axis=1)[None, :]
    csq_sem = jnp.sum(cb_sem ** 2, axis=1)[None, :]
    zfsq = jnp.sum(zf ** 2, axis=1, keepdims=True)
    zssq = jnp.sum(zs ** 2, axis=1, keepdims=True)
    b_syn = b_ctx_syn[None, :]
    b_sem = b_ctx_sem[None, :]

    nb = B // BLOCK_B
    row_spec = pl.BlockSpec((BLOCK_B, DIM), lambda i: (i, 0))
    full = lambda shape: pl.BlockSpec(shape, lambda i: (0,) * len(shape))

    out_shapes = (
        jax.ShapeDtypeStruct((B, DIM), jnp.float32),
        jax.ShapeDtypeStruct((B, DIM), jnp.float32),
        jax.ShapeDtypeStruct((B, 1), jnp.int32),
        jax.ShapeDtypeStruct((B, 1), jnp.int32),
        jax.ShapeDtypeStruct((1, 1), jnp.float32),
    )
    out_specs = (
        row_spec,
        row_spec,
        pl.BlockSpec((BLOCK_B, 1), lambda i: (i, 0)),
        pl.BlockSpec((BLOCK_B, 1), lambda i: (i, 0)),
        pl.BlockSpec((1, 1), lambda i: (0, 0)),
    )
    sq_spec = pl.BlockSpec((BLOCK_B, 1), lambda i: (i, 0))
    in_specs = [
        row_spec, row_spec, sq_spec, sq_spec,
        full((DIM, N_SYN)), full((N_SYN, DIM)), full((DIM, N_SYN)),
        full((1, N_SYN)), full((1, N_SYN)),
        full((DIM, N_SEM)), full((N_SEM, DIM)), full((DIM, N_SEM)),
        full((1, N_SEM)), full((1, N_SEM)),
    ]

    qf, qs, idx_syn, idx_sem, loss_acc = pl.pallas_call(
        _fused_kernel,
        grid=(nb,),
        in_specs=in_specs,
        out_specs=out_specs,
        out_shape=out_shapes,
    )(zf, zs, zfsq, zssq, cbT_syn, cb_syn, WT_syn, b_syn, csq_syn,
      cbT_sem, cb_sem, WT_sem, b_sem, csq_sem)

    zq_syn = jax.lax.complex(qf[:, :D], qf[:, D:])
    zq_sem = jax.lax.complex(qs[:, :D], qs[:, D:])
    loss = loss_acc[0, 0] * ((1.0 + COMMITMENT_COST) / (B * DIM))
    return (zq_syn, zq_sem, loss, idx_syn[:, 0], idx_sem[:, 0])


# speedup vs baseline: 1.2979x; 1.2867x over previous
import functools

import jax
import jax.numpy as jnp
from jax.experimental import pallas as pl

B = 16384
D = 128
DIM = 2 * D
N_SYN = 512
N_SEM = 1024
CTX_GATE_STRENGTH = 2.0
COMMITMENT_COST = 0.25

BLOCK_B = 512


def _vq_block(z, zsq, cbT, cb, WT, b, csq):
    k = cb.shape[0]
    zc = jax.lax.dot_general(z, cbT, (((1,), (0,)), ((), ())),
                             preferred_element_type=jnp.float32)
    d = (zsq + csq) - 2.0 * zc
    logits = jax.lax.dot_general(z, WT, (((1,), (0,)), ((), ())),
                                 preferred_element_type=jnp.float32) + b
    m = jnp.max(logits, axis=1, keepdims=True)
    e = jnp.exp(logits - m)
    bias = CTX_GATE_STRENGTH * (e / jnp.sum(e, axis=1, keepdims=True))
    dtot = d - bias
    dmin = jnp.min(dtot, axis=1, keepdims=True)
    lane = jax.lax.broadcasted_iota(jnp.int32, dtot.shape, 1)
    idx = jnp.min(jnp.where(dtot == dmin, lane, k), axis=1)
    onehot = (lane == idx[:, None]).astype(jnp.float32)
    zq = jax.lax.dot_general(onehot, cb, (((1,), (0,)), ((), ())),
                             preferred_element_type=jnp.float32)
    r = zq - z
    return zq, idx, jnp.sum(r * r)


def _fused_kernel(zf_ref, zs_ref, zfsq_ref, zssq_ref,
                  cbT_syn_ref, cb_syn_ref, WT_syn_ref, b_syn_ref, csq_syn_ref,
                  cbT_sem_ref, cb_sem_ref, WT_sem_ref, b_sem_ref, csq_sem_ref,
                  qf_ref, qs_ref, idx_syn_ref, idx_sem_ref, loss_ref):
    qf, i_syn, l_syn = _vq_block(zf_ref[...], zfsq_ref[...],
                                 cbT_syn_ref[...], cb_syn_ref[...],
                                 WT_syn_ref[...], b_syn_ref[...], csq_syn_ref[...])
    qs, i_sem, l_sem = _vq_block(zs_ref[...], zssq_ref[...],
                                 cbT_sem_ref[...], cb_sem_ref[...],
                                 WT_sem_ref[...], b_sem_ref[...], csq_sem_ref[...])
    qf_ref[...] = qf
    qs_ref[...] = qs
    idx_syn_ref[...] = i_syn[:, None]
    idx_sem_ref[...] = i_sem[:, None]

    @pl.when(pl.program_id(0) == 0)
    def _init():
        loss_ref[...] = jnp.zeros_like(loss_ref)

    loss_ref[...] += l_syn + l_sem


@functools.partial(jax.jit, static_argnames=())
def kernel(z_fast_real, z_fast_imag, z_slow_real, z_slow_imag,
           cb_syn, cb_sem, W_ctx_syn, b_ctx_syn, W_ctx_sem, b_ctx_sem):
    zf = jnp.concatenate([z_fast_real, z_fast_imag], axis=1)
    zs = jnp.concatenate([z_slow_real, z_slow_imag], axis=1)
    cbT_syn = cb_syn.T
    cbT_sem = cb_sem.T
    WT_syn = W_ctx_syn.T
    WT_sem = W_ctx_sem.T
    csq_syn = jnp.sum(cb_syn ** 2, axis=1)[None, :]
    csq_sem = jnp.sum(cb_sem ** 2, axis=1)[None, :]
    zfsq = jnp.sum(zf ** 2, axis=1, keepdims=True)
    zssq = jnp.sum(zs ** 2, axis=1, keepdims=True)
    b_syn = b_ctx_syn[None, :]
    b_sem = b_ctx_sem[None, :]

    nb = B // BLOCK_B
    row_spec = pl.BlockSpec((BLOCK_B, DIM), lambda i: (i, 0))
    full = lambda shape: pl.BlockSpec(shape, lambda i: (0,) * len(shape))

    out_shapes = (
        jax.ShapeDtypeStruct((B, DIM), jnp.float32),
        jax.ShapeDtypeStruct((B, DIM), jnp.float32),
        jax.ShapeDtypeStruct((B, 1), jnp.int32),
        jax.ShapeDtypeStruct((B, 1), jnp.int32),
        jax.ShapeDtypeStruct((1, 1), jnp.float32),
    )
    out_specs = (
        row_spec,
        row_spec,
        pl.BlockSpec((BLOCK_B, 1), lambda i: (i, 0)),
        pl.BlockSpec((BLOCK_B, 1), lambda i: (i, 0)),
        pl.BlockSpec((1, 1), lambda i: (0, 0)),
    )
    sq_spec = pl.BlockSpec((BLOCK_B, 1), lambda i: (i, 0))
    in_specs = [
        row_spec, row_spec, sq_spec, sq_spec,
        full((DIM, N_SYN)), full((N_SYN, DIM)), full((DIM, N_SYN)),
        full((1, N_SYN)), full((1, N_SYN)),
        full((DIM, N_SEM)), full((N_SEM, DIM)), full((DIM, N_SEM)),
        full((1, N_SEM)), full((1, N_SEM)),
    ]

    qf, qs, idx_syn, idx_sem, loss_acc = pl.pallas_call(
        _fused_kernel,
        grid=(nb,),
        in_specs=in_specs,
        out_specs=out_specs,
        out_shape=out_shapes,
    )(zf, zs, zfsq, zssq, cbT_syn, cb_syn, WT_syn, b_syn, csq_syn,
      cbT_sem, cb_sem, WT_sem, b_sem, csq_sem)

    zq_syn = jax.lax.complex(qf[:, :D], qf[:, D:])
    zq_sem = jax.lax.complex(qs[:, :D], qs[:, D:])
    loss = loss_acc[0, 0] * ((1.0 + COMMITMENT_COST) / (B * DIM))
    return (zq_syn, zq_sem, loss, idx_syn[:, 0], idx_sem[:, 0])


# speedup vs baseline: 1.4154x; 1.0906x over previous
import jax
import jax.numpy as jnp
from jax.experimental import pallas as pl

B = 16384
D = 128
DIM = 2 * D
N_SYN = 512
N_SEM = 1024
CTX_GATE_STRENGTH = 2.0
COMMITMENT_COST = 0.25

BLOCK_B = 1024


def _vq_block(z, zsq, cbT, cb, WT, b, csq):
    k = cb.shape[0]
    zc = jax.lax.dot_general(z, cbT, (((1,), (0,)), ((), ())),
                             preferred_element_type=jnp.float32)
    d = (zsq + csq) - 2.0 * zc
    logits = jax.lax.dot_general(z, WT, (((1,), (0,)), ((), ())),
                                 preferred_element_type=jnp.float32) + b
    m = jnp.max(logits, axis=1, keepdims=True)
    e = jnp.exp(logits - m)
    bias = CTX_GATE_STRENGTH * (e / jnp.sum(e, axis=1, keepdims=True))
    dtot = d - bias
    dmin = jnp.min(dtot, axis=1, keepdims=True)
    lane = jax.lax.broadcasted_iota(jnp.int32, dtot.shape, 1)
    idx = jnp.min(jnp.where(dtot == dmin, lane, k), axis=1)
    onehot = (lane == idx[:, None]).astype(jnp.float32)
    zq = jax.lax.dot_general(onehot, cb, (((1,), (0,)), ((), ())),
                             preferred_element_type=jnp.float32)
    r = zq - z
    return zq, idx, jnp.sum(r * r)


def _fused_kernel(zfr_ref, zfi_ref, zsr_ref, zsi_ref, zfsq_ref, zssq_ref,
                  cbT_syn_ref, cb_syn_ref, WT_syn_ref, b_syn_ref, csq_syn_ref,
                  cbT_sem_ref, cb_sem_ref, WT_sem_ref, b_sem_ref, csq_sem_ref,
                  qf_ref, qs_ref, idx_syn_ref, idx_sem_ref, loss_ref):
    zf = jnp.concatenate([zfr_ref[...], zfi_ref[...]], axis=1)
    zs = jnp.concatenate([zsr_ref[...], zsi_ref[...]], axis=1)
    qf, i_syn, l_syn = _vq_block(zf, zfsq_ref[...],
                                 cbT_syn_ref[...], cb_syn_ref[...],
                                 WT_syn_ref[...], b_syn_ref[...], csq_syn_ref[...])
    qs, i_sem, l_sem = _vq_block(zs, zssq_ref[...],
                                 cbT_sem_ref[...], cb_sem_ref[...],
                                 WT_sem_ref[...], b_sem_ref[...], csq_sem_ref[...])
    qf_ref[...] = qf
    qs_ref[...] = qs
    idx_syn_ref[...] = i_syn[:, None]
    idx_sem_ref[...] = i_sem[:, None]

    @pl.when(pl.program_id(0) == 0)
    def _init():
        loss_ref[...] = jnp.zeros_like(loss_ref)

    loss_ref[...] += l_syn + l_sem


def kernel(z_fast_real, z_fast_imag, z_slow_real, z_slow_imag,
           cb_syn, cb_sem, W_ctx_syn, b_ctx_syn, W_ctx_sem, b_ctx_sem):
    cbT_syn = cb_syn.T
    cbT_sem = cb_sem.T
    WT_syn = W_ctx_syn.T
    WT_sem = W_ctx_sem.T
    csq_syn = jnp.sum(cb_syn ** 2, axis=1)[None, :]
    csq_sem = jnp.sum(cb_sem ** 2, axis=1)[None, :]
    zfsq = jnp.sum(jnp.concatenate([z_fast_real, z_fast_imag], axis=1) ** 2,
                   axis=1, keepdims=True)
    zssq = jnp.sum(jnp.concatenate([z_slow_real, z_slow_imag], axis=1) ** 2,
                   axis=1, keepdims=True)
    b_syn = b_ctx_syn[None, :]
    b_sem = b_ctx_sem[None, :]

    nb = B // BLOCK_B
    half_spec = pl.BlockSpec((BLOCK_B, D), lambda i: (i, 0))
    row_spec = pl.BlockSpec((BLOCK_B, DIM), lambda i: (i, 0))
    sq_spec = pl.BlockSpec((BLOCK_B, 1), lambda i: (i, 0))
    full = lambda shape: pl.BlockSpec(shape, lambda i: (0,) * len(shape))

    out_shapes = (
        jax.ShapeDtypeStruct((B, DIM), jnp.float32),
        jax.ShapeDtypeStruct((B, DIM), jnp.float32),
        jax.ShapeDtypeStruct((B, 1), jnp.int32),
        jax.ShapeDtypeStruct((B, 1), jnp.int32),
        jax.ShapeDtypeStruct((1, 1), jnp.float32),
    )
    out_specs = (
        row_spec,
        row_spec,
        sq_spec,
        sq_spec,
        pl.BlockSpec((1, 1), lambda i: (0, 0)),
    )
    in_specs = [
        half_spec, half_spec, half_spec, half_spec, sq_spec, sq_spec,
        full((DIM, N_SYN)), full((N_SYN, DIM)), full((DIM, N_SYN)),
        full((1, N_SYN)), full((1, N_SYN)),
        full((DIM, N_SEM)), full((N_SEM, DIM)), full((DIM, N_SEM)),
        full((1, N_SEM)), full((1, N_SEM)),
    ]

    qf, qs, idx_syn, idx_sem, loss_acc = pl.pallas_call(
        _fused_kernel,
        grid=(nb,),
        in_specs=in_specs,
        out_specs=out_specs,
        out_shape=out_shapes,
    )(z_fast_real, z_fast_imag, z_slow_real, z_slow_imag, zfsq, zssq,
      cbT_syn, cb_syn, WT_syn, b_syn, csq_syn,
      cbT_sem, cb_sem, WT_sem, b_sem, csq_sem)

    zq_syn = jax.lax.complex(qf[:, :D], qf[:, D:])
    zq_sem = jax.lax.complex(qs[:, :D], qs[:, D:])
    loss = loss_acc[0, 0] * ((1.0 + COMMITMENT_COST) / (B * DIM))
    return (zq_syn, zq_sem, loss, idx_syn[:, 0], idx_sem[:, 0])


# speedup vs baseline: 1.4333x; 1.0127x over previous
import jax
import jax.numpy as jnp
from jax.experimental import pallas as pl

B = 16384
D = 128
DIM = 2 * D
N_SYN = 512
N_SEM = 1024
CTX_GATE_STRENGTH = 2.0
COMMITMENT_COST = 0.25

BLOCK_B = 2048


def _vq_block(z, zsq, cbT, cb, WT, b, csq):
    k = cb.shape[0]
    zc = jax.lax.dot_general(z, cbT, (((1,), (0,)), ((), ())),
                             preferred_element_type=jnp.float32)
    d = (zsq + csq) - 2.0 * zc
    logits = jax.lax.dot_general(z, WT, (((1,), (0,)), ((), ())),
                                 preferred_element_type=jnp.float32) + b
    m = jnp.max(logits, axis=1, keepdims=True)
    e = jnp.exp(logits - m)
    bias = CTX_GATE_STRENGTH * (e / jnp.sum(e, axis=1, keepdims=True))
    dtot = d - bias
    dmin = jnp.min(dtot, axis=1, keepdims=True)
    lane = jax.lax.broadcasted_iota(jnp.int32, dtot.shape, 1)
    idx = jnp.min(jnp.where(dtot == dmin, lane, k), axis=1)
    onehot = (lane == idx[:, None]).astype(jnp.float32)
    zq = jax.lax.dot_general(onehot, cb, (((1,), (0,)), ((), ())),
                             preferred_element_type=jnp.float32)
    r = zq - z
    return zq, idx, jnp.sum(r * r)


def _fused_kernel(zfr_ref, zfi_ref, zsr_ref, zsi_ref, zfsq_ref, zssq_ref,
                  cbT_syn_ref, cb_syn_ref, WT_syn_ref, b_syn_ref, csq_syn_ref,
                  cbT_sem_ref, cb_sem_ref, WT_sem_ref, b_sem_ref, csq_sem_ref,
                  qf_ref, qs_ref, idx_syn_ref, idx_sem_ref, loss_ref):
    zf = jnp.concatenate([zfr_ref[...], zfi_ref[...]], axis=1)
    zs = jnp.concatenate([zsr_ref[...], zsi_ref[...]], axis=1)
    qf, i_syn, l_syn = _vq_block(zf, zfsq_ref[...],
                                 cbT_syn_ref[...], cb_syn_ref[...],
                                 WT_syn_ref[...], b_syn_ref[...], csq_syn_ref[...])
    qs, i_sem, l_sem = _vq_block(zs, zssq_ref[...],
                                 cbT_sem_ref[...], cb_sem_ref[...],
                                 WT_sem_ref[...], b_sem_ref[...], csq_sem_ref[...])
    qf_ref[...] = qf
    qs_ref[...] = qs
    idx_syn_ref[...] = i_syn[:, None]
    idx_sem_ref[...] = i_sem[:, None]

    @pl.when(pl.program_id(0) == 0)
    def _init():
        loss_ref[...] = jnp.zeros_like(loss_ref)

    loss_ref[...] += l_syn + l_sem


def kernel(z_fast_real, z_fast_imag, z_slow_real, z_slow_imag,
           cb_syn, cb_sem, W_ctx_syn, b_ctx_syn, W_ctx_sem, b_ctx_sem):
    cbT_syn = cb_syn.T
    cbT_sem = cb_sem.T
    WT_syn = W_ctx_syn.T
    WT_sem = W_ctx_sem.T
    csq_syn = jnp.sum(cb_syn ** 2, axis=1)[None, :]
    csq_sem = jnp.sum(cb_sem ** 2, axis=1)[None, :]
    zfsq = jnp.sum(jnp.concatenate([z_fast_real, z_fast_imag], axis=1) ** 2,
                   axis=1, keepdims=True)
    zssq = jnp.sum(jnp.concatenate([z_slow_real, z_slow_imag], axis=1) ** 2,
                   axis=1, keepdims=True)
    b_syn = b_ctx_syn[None, :]
    b_sem = b_ctx_sem[None, :]

    nb = B // BLOCK_B
    half_spec = pl.BlockSpec((BLOCK_B, D), lambda i: (i, 0))
    row_spec = pl.BlockSpec((BLOCK_B, DIM), lambda i: (i, 0))
    sq_spec = pl.BlockSpec((BLOCK_B, 1), lambda i: (i, 0))
    full = lambda shape: pl.BlockSpec(shape, lambda i: (0,) * len(shape))

    out_shapes = (
        jax.ShapeDtypeStruct((B, DIM), jnp.float32),
        jax.ShapeDtypeStruct((B, DIM), jnp.float32),
        jax.ShapeDtypeStruct((B, 1), jnp.int32),
        jax.ShapeDtypeStruct((B, 1), jnp.int32),
        jax.ShapeDtypeStruct((1, 1), jnp.float32),
    )
    out_specs = (
        row_spec,
        row_spec,
        sq_spec,
        sq_spec,
        pl.BlockSpec((1, 1), lambda i: (0, 0)),
    )
    in_specs = [
        half_spec, half_spec, half_spec, half_spec, sq_spec, sq_spec,
        full((DIM, N_SYN)), full((N_SYN, DIM)), full((DIM, N_SYN)),
        full((1, N_SYN)), full((1, N_SYN)),
        full((DIM, N_SEM)), full((N_SEM, DIM)), full((DIM, N_SEM)),
        full((1, N_SEM)), full((1, N_SEM)),
    ]

    qf, qs, idx_syn, idx_sem, loss_acc = pl.pallas_call(
        _fused_kernel,
        grid=(nb,),
        in_specs=in_specs,
        out_specs=out_specs,
        out_shape=out_shapes,
    )(z_fast_real, z_fast_imag, z_slow_real, z_slow_imag, zfsq, zssq,
      cbT_syn, cb_syn, WT_syn, b_syn, csq_syn,
      cbT_sem, cb_sem, WT_sem, b_sem, csq_sem)

    zq_syn = jax.lax.complex(qf[:, :D], qf[:, D:])
    zq_sem = jax.lax.complex(qs[:, :D], qs[:, D:])
    loss = loss_acc[0, 0] * ((1.0 + COMMITMENT_COST) / (B * DIM))
    return (zq_syn, zq_sem, loss, idx_syn[:, 0], idx_sem[:, 0])
